# baseline (speedup 1.0000x reference)
import functools

import jax
import jax.numpy as jnp
from jax import lax
from jax.experimental import pallas as pl
from jax.experimental.pallas import tpu as pltpu
from jax.experimental.pallas import tpu_sc as plsc

NC = 2
NS = 16
NW = NC * NS
L = 16
N_TAB = 1000000
PACK = 4
LANES = 128

B = 16384
K = 32
B_PER_W = B // NW
N_SUB = 4
SUB = B_PER_W // N_SUB
N_HALF = 2
HALF = B_PER_W // N_HALF


def _body(u_hbm, i_hbm, p_hbm, q_hbm, out_hbm,
          ui_v, ii_v, ut_v, it_v, pu_v, qi_v, out_v, sem_p, sem_q):
    wid = lax.axis_index("s") * NC + lax.axis_index("c")
    base = wid * B_PER_W

    pltpu.sync_copy(u_hbm.at[wid], ui_v)
    pltpu.sync_copy(i_hbm.at[wid], ii_v)

    def shift(j, carry):
        def inner(v, carry2):
            s = pl.ds(v * L, L)
            ut_v[j, s] = jax.lax.shift_right_logical(ui_v[j, s], 2)
            it_v[j, s] = jax.lax.shift_right_logical(ii_v[j, s], 2)
            return carry2
        return lax.fori_loop(0, SUB // L, inner, carry)
    lax.fori_loop(0, N_SUB, shift, 0, unroll=False)

    for h in range(N_HALF):
        copies = []
        for j in range(HALF // SUB):
            rows = pl.ds(j * SUB, SUB)
            g = h * (HALF // SUB) + j
            copies.append(pltpu.async_copy(
                p_hbm.at[ut_v.at[g]], pu_v.at[rows], sem_p))
            copies.append(pltpu.async_copy(
                q_hbm.at[it_v.at[g]], qi_v.at[rows], sem_q))
        for c in copies:
            c.wait()

        def chunk(c, carry):
            rows = c * L + lax.iota(jnp.int32, L)
            gj = h * (HALF // SUB) + c * L // SUB
            s = pl.ds((c * L) % SUB, L)
            cu = jax.lax.shift_left(ui_v[gj, s] & 3, 5)
            ci = jax.lax.shift_left(ii_v[gj, s] & 3, 5)
            acc = jnp.zeros((L,), jnp.float32)
            for k in range(K):
                acc = acc + (plsc.load_gather(pu_v, [rows, cu + k]) *
                             plsc.load_gather(qi_v, [rows, ci + k]))
            out_v[pl.ds(h * HALF + c * L, L)] = acc
            return carry

        lax.fori_loop(0, HALF // L, chunk, 0, unroll=False)

    pltpu.sync_copy(out_v, out_hbm.at[pl.ds(base, B_PER_W)])


@jax.jit
def _funk_svd_sc(u2, i2, P4, Q4):
    mesh = plsc.VectorSubcoreMesh(core_axis_name="c", subcore_axis_name="s")
    return pl.kernel(
        _body,
        out_type=jax.ShapeDtypeStruct((B,), jnp.float32),
        mesh=mesh,
        scratch_types=[
            pltpu.VMEM((N_SUB, SUB), jnp.int32),
            pltpu.VMEM((N_SUB, SUB), jnp.int32),
            pltpu.VMEM((N_SUB, SUB), jnp.int32),
            pltpu.VMEM((N_SUB, SUB), jnp.int32),
            pltpu.VMEM((HALF, LANES), jnp.float32),
            pltpu.VMEM((HALF, LANES), jnp.float32),
            pltpu.VMEM((B_PER_W,), jnp.float32),
            pltpu.SemaphoreType.DMA,
            pltpu.SemaphoreType.DMA,
        ],
        compiler_params=pltpu.CompilerParams(
            needs_layout_passes=False, use_tc_tiling_on_sc=True),
    )(u2, i2, P4, Q4)


def kernel(u, i, P, Q):
    u2 = u.astype(jnp.int32).reshape(NW, N_SUB, SUB)
    i2 = i.astype(jnp.int32).reshape(NW, N_SUB, SUB)
    P4 = P.reshape(N_TAB // PACK, LANES)
    Q4 = Q.reshape(N_TAB // PACK, LANES)
    return _funk_svd_sc(u2, i2, P4, Q4)


# speedup vs baseline: 1.0058x; 1.0058x over previous
import functools

import jax
import jax.numpy as jnp
from jax import lax
from jax.experimental import pallas as pl
from jax.experimental.pallas import tpu as pltpu
from jax.experimental.pallas import tpu_sc as plsc

NC = 2
NS = 16
NW = NC * NS
L = 16

B = 16384
K = 32
B_PER_W = B // NW
N_SUB = 4
SUB = B_PER_W // N_SUB


def _body(u_hbm, i_hbm, p_hbm, q_hbm, out_hbm,
          ui_v, ii_v, pu_v, qi_v, out_v, sem_p, sem_q):
    wid = lax.axis_index("s") * NC + lax.axis_index("c")
    base = wid * B_PER_W

    pltpu.sync_copy(u_hbm.at[wid], ui_v)
    pltpu.sync_copy(i_hbm.at[wid], ii_v)

    copies = []
    for j in range(N_SUB):
        rows = pl.ds(j * SUB, SUB)
        copies.append(pltpu.async_copy(p_hbm.at[ui_v.at[j]], pu_v.at[rows], sem_p))
        copies.append(pltpu.async_copy(q_hbm.at[ii_v.at[j]], qi_v.at[rows], sem_q))
    for c in copies:
        c.wait()

    def chunk(c, carry):
        rows = c * L + lax.iota(jnp.int32, L)
        acc = jnp.zeros((L,), jnp.float32)
        for k in range(K):
            col = jnp.full((L,), k, jnp.int32)
            acc = acc + (plsc.load_gather(pu_v, [rows, col]) *
                         plsc.load_gather(qi_v, [rows, col]))
        out_v[pl.ds(c * L, L)] = acc
        return carry

    lax.fori_loop(0, B_PER_W // L, chunk, 0, unroll=False)

    pltpu.sync_copy(out_v, out_hbm.at[pl.ds(base, B_PER_W)])


@jax.jit
def _funk_svd_sc(u2, i2, P, Q):
    mesh = plsc.VectorSubcoreMesh(core_axis_name="c", subcore_axis_name="s")
    return pl.kernel(
        _body,
        out_type=jax.ShapeDtypeStruct((B,), jnp.float32),
        mesh=mesh,
        scratch_types=[
            pltpu.VMEM((N_SUB, SUB), jnp.int32),
            pltpu.VMEM((N_SUB, SUB), jnp.int32),
            pltpu.VMEM((B_PER_W, K), jnp.float32),
            pltpu.VMEM((B_PER_W, K), jnp.float32),
            pltpu.VMEM((B_PER_W,), jnp.float32),
            pltpu.SemaphoreType.DMA,
            pltpu.SemaphoreType.DMA,
        ],
        compiler_params=pltpu.CompilerParams(
            needs_layout_passes=False, use_tc_tiling_on_sc=False),
    )(u2, i2, P, Q)


def kernel(u, i, P, Q):
    u2 = u.astype(jnp.int32).reshape(NW, N_SUB, SUB)
    i2 = i.astype(jnp.int32).reshape(NW, N_SUB, SUB)
    return _funk_svd_sc(u2, i2, P, Q)


# speedup vs baseline: 1.4954x; 1.4867x over previous
import functools

import jax
import jax.numpy as jnp
from jax import lax
from jax.experimental import pallas as pl
from jax.experimental.pallas import tpu as pltpu
from jax.experimental.pallas import tpu_sc as plsc

NC = 2
NS = 16
NW = NC * NS
L = 16

B = 16384
K = 32
B_PER_W = B // NW
N_SUB = 4
SUB = B_PER_W // N_SUB
NPH = 2
H = B_PER_W // NPH


def _body(idx_hbm, p_hbm, q_hbm, out_hbm,
          idx_v, pu_v, qi_v, out_v, sem_p, sem_q):
    wid = lax.axis_index("s") * NC + lax.axis_index("c")
    base = wid * B_PER_W

    pltpu.sync_copy(idx_hbm.at[wid], idx_v)

    for ph in range(NPH):
        def issue(g, carry, ph=ph):
            g = g + ph * (H // L)
            j = g // (SUB // L)
            l0 = (g % (SUB // L)) * L
            uv = idx_v[j, pl.ds(l0, L)]
            iv = idx_v[N_SUB + j, pl.ds(l0, L)]
            for e in range(L):
                r = (g - ph * (H // L)) * L + e
                pltpu.async_copy(
                    p_hbm.at[pl.ds(uv[e], 1)], pu_v.at[pl.ds(r, 1)], sem_p)
                pltpu.async_copy(
                    q_hbm.at[pl.ds(iv[e], 1)], qi_v.at[pl.ds(r, 1)], sem_q)
            return carry

        lax.fori_loop(0, H // L, issue, 0, unroll=False)

        def drain(r, carry):
            pltpu.make_async_copy(
                p_hbm.at[pl.ds(0, 1)], pu_v.at[pl.ds(0, 1)], sem_p).wait()
            pltpu.make_async_copy(
                q_hbm.at[pl.ds(0, 1)], qi_v.at[pl.ds(0, 1)], sem_q).wait()
            return carry

        lax.fori_loop(0, H, drain, 0, unroll=False)

        def chunk(c, carry, ph=ph):
            rows = c * L + lax.iota(jnp.int32, L)
            acc = jnp.zeros((L,), jnp.float32)
            for k in range(K):
                col = jnp.full((L,), k, jnp.int32)
                acc = acc + (plsc.load_gather(pu_v, [rows, col]) *
                             plsc.load_gather(qi_v, [rows, col]))
            out_v[pl.ds(ph * H + c * L, L)] = acc
            return carry

        lax.fori_loop(0, H // L, chunk, 0, unroll=False)

    pltpu.sync_copy(out_v, out_hbm.at[pl.ds(base, B_PER_W)])


@jax.jit
def _funk_svd_sc(idx2, P, Q):
    mesh = plsc.VectorSubcoreMesh(core_axis_name="c", subcore_axis_name="s")
    return pl.kernel(
        _body,
        out_type=jax.ShapeDtypeStruct((B,), jnp.float32),
        mesh=mesh,
        scratch_types=[
            pltpu.VMEM((2 * N_SUB, SUB), jnp.int32),
            pltpu.VMEM((H, K), jnp.float32),
            pltpu.VMEM((H, K), jnp.float32),
            pltpu.VMEM((B_PER_W,), jnp.float32),
            pltpu.SemaphoreType.DMA,
            pltpu.SemaphoreType.DMA,
        ],
        compiler_params=pltpu.CompilerParams(
            needs_layout_passes=False, use_tc_tiling_on_sc=True),
    )(idx2, P, Q)


def kernel(u, i, P, Q):
    u2 = u.astype(jnp.int32).reshape(NW, N_SUB, SUB)
    i2 = i.astype(jnp.int32).reshape(NW, N_SUB, SUB)
    idx2 = jnp.concatenate([u2, i2], axis=1)
    return _funk_svd_sc(idx2, P, Q)
